# Initial kernel scaffold; baseline (speedup 1.0000x reference)
import jax
import jax.numpy as jnp
from jax.experimental import pallas as pl


def kernel(head_ids, offsets, table):
    raise NotImplementedError("write your pallas kernel here")


# speedup vs baseline: 3.0164x; 3.0164x over previous
import jax
import jax.numpy as jnp
from jax import lax
from jax.experimental import pallas as pl
from jax.experimental.pallas import tpu as pltpu
from jax.experimental.pallas import tpu_sc as plsc

_NC = 2
_NS = 16
_NW = _NC * _NS
_LANES = 16

_STRIP = 128
_STRIPS_PER_CHUNK = 13
_CHUNK = _STRIP * _STRIPS_PER_CHUNK


def _build_gather(N, H, D, n_off_pad):
    assert N % _NW == 0
    per_w = N // _NW
    assert per_w % _CHUNK == 0
    chunks = per_w // _CHUNK

    mesh = plsc.VectorSubcoreMesh(core_axis_name="c", subcore_axis_name="s")

    @pl.kernel(
        out_type=jax.ShapeDtypeStruct((N, D), jnp.float32),
        mesh=mesh,
        compiler_params=pltpu.CompilerParams(
            needs_layout_passes=False, use_tc_tiling_on_sc=False),
        scratch_types=[
            pltpu.VMEM((n_off_pad,), jnp.int32),
            pltpu.VMEM((2, _CHUNK), jnp.int32),
            pltpu.VMEM((2, _CHUNK, D), jnp.float32),
            pltpu.SemaphoreType.DMA,
            pltpu.SemaphoreType.DMA,
            pltpu.SemaphoreType.DMA,
            pltpu.SemaphoreType.DMA,
        ],
    )
    def gather_kernel(ids_hbm, offs_hbm, table_hbm, out_hbm,
                      offs_v, idx_v, rows_v, gsem_a, gsem_b, osem_a, osem_b):
        wid = lax.axis_index("s") * _NC + lax.axis_index("c")
        base = wid * per_w

        pltpu.sync_copy(offs_hbm, offs_v)
        iota = lax.broadcasted_iota(jnp.int32, (_LANES,), 0)

        gsems = [gsem_a, gsem_b]
        osems = [osem_a, osem_b]
        pending_gather = [None, None]
        pending_out = [None, None]

        def issue(g):
            b = g % 2
            if pending_out[b] is not None:
                pending_out[b].wait()
                pending_out[b] = None
            start = base + g * _CHUNK
            pltpu.sync_copy(ids_hbm.at[pl.ds(start, _CHUNK)], idx_v.at[b])

            def add_offsets(j, carry):
                lane0 = pl.multiple_of(j * _LANES, _LANES)
                pos = start + j * _LANES + iota
                off = plsc.load_gather(offs_v, [lax.rem(pos, H)])
                idx_v[b, pl.ds(lane0, _LANES)] = (
                    idx_v[b, pl.ds(lane0, _LANES)] + off)
                return carry

            lax.fori_loop(0, _CHUNK // _LANES, add_offsets, 0)

            copies = []
            for k in range(_STRIPS_PER_CHUNK):
                c = pltpu.async_copy(
                    table_hbm.at[idx_v.at[b, pl.ds(k * _STRIP, _STRIP)]],
                    rows_v.at[b, pl.ds(k * _STRIP, _STRIP)],
                    gsems[b])
                copies.append(c)
            pending_gather[b] = (g, copies)

        def drain(b):
            g, copies = pending_gather[b]
            for c in copies:
                c.wait()
            pending_gather[b] = None
            pending_out[b] = pltpu.async_copy(
                rows_v.at[b],
                out_hbm.at[pl.ds(base + g * _CHUNK, _CHUNK)],
                osems[b])

        issue(0)
        for g in range(1, chunks):
            issue(g)
            drain((g - 1) % 2)
        drain((chunks - 1) % 2)
        for b in range(2):
            if pending_out[b] is not None:
                pending_out[b].wait()

    return gather_kernel


def kernel(head_ids, offsets, table):
    B, S, H = head_ids.shape
    V, D = table.shape
    N = B * S * H
    n_off_pad = 128
    ids = head_ids.reshape(N).astype(jnp.int32)
    offs = jnp.zeros((n_off_pad,), jnp.int32).at[:H].set(
        offsets.astype(jnp.int32))
    out = _build_gather(N, H, D, n_off_pad)(ids, offs, table)
    return out.reshape(B, S, H, D)


# speedup vs baseline: 4.4155x; 1.4638x over previous
import jax
import jax.numpy as jnp
from jax import lax
from jax.experimental import pallas as pl
from jax.experimental.pallas import tpu as pltpu
from jax.experimental.pallas import tpu_sc as plsc

_NC = 2
_NS = 16
_NW = _NC * _NS
_LANES = 16

_BLK = 128


def _build_gather(S, H, D, n_off_pad):
    NSH = S * H
    blocks = NSH * 8
    assert blocks % _NW == 0
    per_w = blocks // _NW
    n_ids_w = per_w * _BLK
    nvec = n_ids_w // _LANES
    DT = D // 8

    mesh = plsc.VectorSubcoreMesh(core_axis_name="c", subcore_axis_name="s")

    @pl.kernel(
        out_type=jax.ShapeDtypeStruct((NSH, DT, 8, 8, 128), jnp.float32),
        mesh=mesh,
        compiler_params=pltpu.CompilerParams(
            needs_layout_passes=False, use_tc_tiling_on_sc=False),
        scratch_types=[
            pltpu.VMEM((n_off_pad,), jnp.int32),
            pltpu.VMEM((n_ids_w,), jnp.int32),
            pltpu.VMEM((2, _BLK, D), jnp.float32),
            pltpu.VMEM((2, DT, 8, 128), jnp.float32),
            pltpu.SemaphoreType.DMA,
            pltpu.SemaphoreType.DMA,
            pltpu.SemaphoreType.DMA,
            pltpu.SemaphoreType.DMA,
        ],
    )
    def gather_kernel(ids_hbm, offs_hbm, table_hbm, o5,
                      offs_v, idx_v, rows_v, tblk_v,
                      gsem_a, gsem_b, osem_a, osem_b):
        wid = lax.axis_index("s") * _NC + lax.axis_index("c")
        base = wid * per_w

        pltpu.sync_copy(offs_hbm, offs_v)
        pltpu.sync_copy(ids_hbm.at[pl.ds(base * _BLK, n_ids_w)], idx_v)
        iota = lax.broadcasted_iota(jnp.int32, (_LANES,), 0)

        @pl.loop(0, nvec, unroll=8)
        def _add_off(j):
            h = lax.rem((base + (j >> 3)) >> 3, H)
            off = plsc.load_gather(offs_v, [jnp.full((_LANES,), 0, jnp.int32) + h])
            sl = pl.ds(j * _LANES, _LANES)
            idx_v[sl] = idx_v[sl] + off

        gsems = [gsem_a, gsem_b]
        osems = [osem_a, osem_b]

        def issue(g, par):
            pltpu.async_copy(
                table_hbm.at[idx_v.at[pl.ds(g * _BLK, _BLK)]],
                rows_v.at[par], gsems[par])

        def finish(g, par, first):
            pltpu.make_async_copy(
                table_hbm.at[pl.ds(0, _BLK)], rows_v.at[par],
                gsems[par]).wait()
            if not first:
                for ti in range(DT):
                    pltpu.make_async_copy(
                        o5.at[0, 0, 0], tblk_v.at[par, ti],
                        osems[par]).wait()
            @pl.loop(0, (_BLK * D) // (_LANES * _LANES), unroll=16)
            def _tr(j):
                ti = j >> 6
                sub = (j >> 3) & 7
                kk = j & 7
                lanes = kk * _LANES + iota
                dims = jnp.full((_LANES,), 0, jnp.int32) + (ti * 8 + sub)
                v = plsc.load_gather(rows_v.at[par], [lanes, dims])
                tblk_v[par, ti, sub, pl.ds(kk * _LANES, _LANES)] = v

            B = base + g
            sh = B >> 3
            tj = lax.rem(B, 8)
            for ti in range(DT):
                pltpu.async_copy(
                    tblk_v.at[par, ti], o5.at[sh, ti, tj], osems[par])

        issue(0, 0)
        issue(1, 1)
        finish(0, 0, True)
        issue(2, 0)
        finish(1, 1, True)

        @pl.loop(0, (per_w - 3) // 2)
        def _blocks(t):
            g = 3 + 2 * t
            issue(g, 1)
            finish(g - 1, 0, False)
            issue(g + 1, 0)
            finish(g, 1, False)

        finish(per_w - 1, 0, False)
        for par in range(2):
            for ti in range(DT):
                pltpu.make_async_copy(
                    o5.at[0, 0, 0], tblk_v.at[par, ti], osems[par]).wait()

    return gather_kernel


def kernel(head_ids, offsets, table):
    B, S, H = head_ids.shape
    V, D = table.shape
    n_off_pad = 128
    ids_t = jnp.transpose(head_ids, (1, 2, 0)).reshape(-1).astype(jnp.int32)
    offs = jnp.zeros((n_off_pad,), jnp.int32).at[:H].set(
        offsets.astype(jnp.int32))
    o5 = _build_gather(S, H, D, n_off_pad)(ids_t, offs, table)
    out = (o5.reshape(S, H, D // 8, 8, 8, 128)
           .transpose(3, 5, 0, 1, 2, 4)
           .reshape(B, S, H, D))
    return out
